# baseline (speedup 1.0000x reference)
import functools

import jax
import jax.numpy as jnp
from jax import lax
from jax.experimental import pallas as pl
from jax.experimental.pallas import tpu as pltpu
from jax.experimental.pallas import tpu_sc as plsc

N = 10000
NPAD = 10240
E = 160000
EPAD = 163840
D = 256
H = 128
NC = 2
NS = 16
BN = 1000
NB = N // BN

_F32 = jnp.float32
_CH = 128
_NHALF = NPAD // NC
_ACC_R = _NHALF + 64
_ACC_W = _NHALF // NS
_ROWS_W = NPAD // NS


def _sc_mesh():
    return plsc.VectorSubcoreMesh(core_axis_name="c", subcore_axis_name="s")


_DEG_NCH = EPAD // (NC * NS) // _CH
_DEG_WIN = 8


@functools.partial(
    pl.kernel,
    out_type=jax.ShapeDtypeStruct((NC * NPAD, 16), _F32),
    mesh=_sc_mesh(),
    scratch_types=[
        pltpu.VMEM((_DEG_NCH, _CH), jnp.int32),
        pltpu.VMEM((_CH, 16), _F32),
        pltpu.VMEM_SHARED((NPAD, 16), _F32),
        pltpu.SemaphoreType.DMA,
    ],
)
def _sc_degree(dst_hbm, zeros_hbm, out_hbm, dst_all, ones_v, acc_sh, sem):
    c = lax.axis_index("c")
    s = lax.axis_index("s")
    w = c * NS + s
    pltpu.sync_copy(dst_hbm.at[w], dst_all)
    for i in range(_CH):
        ones_v[i] = jnp.full((16,), 1.0, _F32)
    pltpu.sync_copy(zeros_hbm.at[pl.ds(s * _ROWS_W, _ROWS_W)],
                    acc_sh.at[pl.ds(s * _ROWS_W, _ROWS_W)])
    plsc.subcore_barrier()

    @pl.loop(0, _DEG_NCH)
    def _(j):
        pltpu.async_copy(ones_v, acc_sh.at[dst_all.at[j]], sem, add=True)

        @pl.when(j >= _DEG_WIN)
        def _():
            pltpu.make_async_copy(ones_v, acc_sh.at[dst_all.at[0]], sem).wait()

    @pl.loop(0, _DEG_WIN)
    def _(j):
        pltpu.make_async_copy(ones_v, acc_sh.at[dst_all.at[0]], sem).wait()

    plsc.subcore_barrier()
    pltpu.sync_copy(acc_sh.at[pl.ds(s * _ROWS_W, _ROWS_W)],
                    out_hbm.at[pl.ds(c * NPAD + s * _ROWS_W, _ROWS_W)])


_SP_NCH = 160
_SP_NBLK = 10


@functools.partial(
    pl.kernel,
    out_type=jax.ShapeDtypeStruct((NPAD, 2, H), _F32),
    mesh=_sc_mesh(),
    scratch_types=[
        pltpu.VMEM((80, _CH), jnp.int32),
        pltpu.VMEM((16, _CH), jnp.int32),
        pltpu.VMEM((2 * 64, 2, H), _F32),
        pltpu.VMEM_SHARED((_ACC_R, 2, H), _F32),
        pltpu.SemaphoreType.DMA,
        pltpu.SemaphoreType.DMA,
    ],
)
def _sc_spmm(hs_hbm, src_hbm, dst_hbm, zeros_hbm, out_hbm,
             dst_all, ring, rows, acc_sh, sem, semi):
    c = lax.axis_index("c")
    s = lax.axis_index("s")
    base = c * _NHALF
    pltpu.sync_copy(dst_hbm.at[s], dst_all)
    pltpu.sync_copy(zeros_hbm.at[pl.ds(s * _ACC_W, _ACC_W)],
                    acc_sh.at[pl.ds(s * _ACC_W, _ACC_W)])

    @pl.when(s == 0)
    def _():
        pltpu.sync_copy(zeros_hbm.at[pl.ds(_NHALF, 64)],
                        acc_sh.at[pl.ds(_NHALF, 64)])

    jnk = _NHALF + lax.rem(lax.iota(jnp.int32, 16) * 4 + s * 2, 64)

    @pl.loop(0, 80)
    def _(r):
        @pl.loop(0, _CH, step=16)
        def _(i):
            v = dst_all[r, pl.ds(i, 16)] - base
            ok = (v >= 0) & (v < _NHALF)
            dst_all[r, pl.ds(i, 16)] = jnp.where(ok, v, jnk + lax.rem(r + i, 2))

    def _gather(j):
        b = j // 16
        rr = lax.rem(b, 2) * 8 + lax.rem(j, 16) // 2
        q = lax.rem(j, 2) * 64
        pltpu.async_copy(hs_hbm.at[ring.at[rr, pl.ds(q, 64)]],
                         rows.at[pl.ds(lax.rem(j, 2) * 64, 64)], sem)

    pltpu.sync_copy(src_hbm.at[s].at[0], ring.at[pl.ds(0, 8)])
    pltpu.async_copy(src_hbm.at[s].at[1], ring.at[pl.ds(8, 8)], semi)
    plsc.subcore_barrier()
    _gather(0)

    @pl.loop(0, _SP_NCH)
    def _(j):
        pltpu.make_async_copy(hs_hbm.at[ring.at[0, pl.ds(0, 64)]],
                              rows.at[pl.ds(0, 64)], sem).wait()

        j1 = j + 1

        @pl.when(j1 < _SP_NCH)
        def _():
            b1 = j1 // 16

            @pl.when(lax.rem(j1, 16) == 0)
            def _():
                pltpu.make_async_copy(src_hbm.at[s].at[0],
                                      ring.at[pl.ds(0, 8)], semi).wait()

                @pl.when(b1 + 1 < _SP_NBLK)
                def _():
                    pltpu.async_copy(
                        src_hbm.at[s].at[b1 + 1],
                        ring.at[pl.ds(lax.rem(b1 + 1, 2) * 8, 8)], semi)

            _gather(j1)

        pltpu.sync_copy(rows.at[pl.ds(lax.rem(j, 2) * 64, 64)],
                        acc_sh.at[dst_all.at[j // 2, pl.ds(lax.rem(j, 2) * 64, 64)]],
                        add=True)

    plsc.subcore_barrier()
    pltpu.sync_copy(acc_sh.at[pl.ds(s * _ACC_W, _ACC_W)],
                    out_hbm.at[pl.ds(base + s * _ACC_W, _ACC_W)])


def _dot(a, b, dims):
    return lax.dot_general(a, b, (dims, ((), ())),
                           precision=lax.Precision.HIGHEST,
                           preferred_element_type=_F32)


def _tab_spec():
    return pl.BlockSpec((BN, 2, H), lambda i: (i, 0, 0))


def _tc_prep_body(deg0_ref, deg1_ref, x_ref, d_ref, xs_ref):
    deg = deg0_ref[0, :, 0:1] + deg1_ref[0, :, 0:1] + 1.0
    d = lax.rsqrt(deg)
    d_ref[...] = d
    xs_ref[...] = (x_ref[...] * d).reshape(BN, 2, H)


def _tc_prep(deg2, x):
    return pl.pallas_call(
        _tc_prep_body,
        grid=(NB,),
        in_specs=[
            pl.BlockSpec((1, BN, 16), lambda i: (0, i, 0)),
            pl.BlockSpec((1, BN, 16), lambda i: (1, i, 0)),
            pl.BlockSpec((BN, D), lambda i: (i, 0)),
        ],
        out_specs=[pl.BlockSpec((BN, 1), lambda i: (i, 0)), _tab_spec()],
        out_shape=[jax.ShapeDtypeStruct((N, 1), _F32),
                   jax.ShapeDtypeStruct((NPAD, 2, H), _F32)],
    )(deg2, deg2, x)


def _tc_layer_body(t_ref, h_ref, d_ref, w_ref, ys_ref):
    d = d_ref[...]
    u = (t_ref[...] + h_ref[...]).reshape(BN, D) * d
    y = jnp.maximum(_dot(u, w_ref[...], ((1,), (0,))), 0.0) * d
    ys_ref[...] = y.reshape(BN, 2, H)


def _tc_layer(t, hs, d, W):
    return pl.pallas_call(
        _tc_layer_body,
        grid=(NB,),
        in_specs=[_tab_spec(), _tab_spec(),
                  pl.BlockSpec((BN, 1), lambda i: (i, 0)),
                  pl.BlockSpec((D, D), lambda i: (0, 0))],
        out_specs=_tab_spec(),
        out_shape=jax.ShapeDtypeStruct((NPAD, 2, H), _F32),
    )(t, hs, d, W)


def _tc_final_body(t_ref, h_ref, d_ref, w_ref, z_ref):
    u = (t_ref[...] + h_ref[...]).reshape(BN, D) * d_ref[...]
    z_ref[...] = _dot(u, w_ref[...], ((1,), (0,)))


def _tc_final(t, hs, d, W):
    return pl.pallas_call(
        _tc_final_body,
        grid=(NB,),
        in_specs=[_tab_spec(), _tab_spec(),
                  pl.BlockSpec((BN, 1), lambda i: (i, 0)),
                  pl.BlockSpec((D, D), lambda i: (0, 0))],
        out_specs=pl.BlockSpec((BN, D), lambda i: (i, 0)),
        out_shape=jax.ShapeDtypeStruct((N, D), _F32),
    )(t, hs, d, W)


def _tc_cos_body(z_ref, o_ref):
    z = z_ref[...]
    zn = z * lax.rsqrt(jnp.sum(z * z, axis=1, keepdims=True))
    g = _dot(zn, zn, ((1,), (1,)))
    o_ref[...] = (jnp.sum(g) * (1.0 / (512.0 * 512.0))).reshape(1, 1)


def _tc_cos(z512):
    return pl.pallas_call(
        _tc_cos_body,
        out_shape=jax.ShapeDtypeStruct((1, 1), _F32),
    )(z512)


def kernel(x, edge_index, W1, W2, W3):
    npad_e = EPAD - E
    src = jnp.concatenate([edge_index[0], jnp.zeros((npad_e,), jnp.int32)])
    dst = jnp.concatenate([edge_index[1],
                           jnp.full((npad_e,), N, jnp.int32)])
    srcR = src.reshape(NS, _SP_NBLK, 8, _CH)
    dstR = dst.reshape(NS, 80, _CH)
    dst_degR = dst.reshape(NC * NS, _DEG_NCH, _CH)
    zeros16 = jnp.zeros((NPAD, 16), _F32)
    zerosT = jnp.zeros((_ACC_R, 2, H), _F32)

    deg2 = _sc_degree(dst_degR, zeros16).reshape(NC, NPAD, 16)
    d, xs = _tc_prep(deg2, x)

    t = _sc_spmm(xs, srcR, dstR, zerosT)
    h1 = _tc_layer(t, xs, d, W1)

    t = _sc_spmm(h1, srcR, dstR, zerosT)
    h2 = _tc_layer(t, h1, d, W2)

    t = _sc_spmm(h2, srcR, dstR, zerosT)
    z = _tc_final(t, h2, d, W3)

    corr = _tc_cos(z[:512])
    return z, corr[0, 0]


# speedup vs baseline: 1.6033x; 1.6033x over previous
import functools

import jax
import jax.numpy as jnp
from jax import lax
from jax.experimental import pallas as pl
from jax.experimental.pallas import tpu as pltpu
from jax.experimental.pallas import tpu_sc as plsc

N = 10000
NPAD = 10240
E = 160000
EPAD = 163840
D = 256
H = 128
NC = 2
NS = 16
BN = 1000
NB = N // BN

_F32 = jnp.float32
_CH = 128
_ROWS_W = NPAD // NS


def _sc_mesh():
    return plsc.VectorSubcoreMesh(core_axis_name="c", subcore_axis_name="s")


_DEG_NCH = EPAD // (NC * NS) // _CH
_DEG_WIN = 8


@functools.partial(
    pl.kernel,
    out_type=jax.ShapeDtypeStruct((NC * NPAD, 16), _F32),
    mesh=_sc_mesh(),
    scratch_types=[
        pltpu.VMEM((_DEG_NCH, _CH), jnp.int32),
        pltpu.VMEM((_CH, 16), _F32),
        pltpu.VMEM_SHARED((NPAD, 16), _F32),
        pltpu.SemaphoreType.DMA,
    ],
)
def _sc_degree(dst_hbm, zeros_hbm, out_hbm, dst_all, ones_v, acc_sh, sem):
    c = lax.axis_index("c")
    s = lax.axis_index("s")
    w = c * NS + s
    pltpu.sync_copy(dst_hbm.at[w], dst_all)
    for i in range(_CH):
        ones_v[i] = jnp.full((16,), 1.0, _F32)
    pltpu.sync_copy(zeros_hbm.at[pl.ds(s * _ROWS_W, _ROWS_W)],
                    acc_sh.at[pl.ds(s * _ROWS_W, _ROWS_W)])
    plsc.subcore_barrier()

    @pl.loop(0, _DEG_NCH)
    def _(j):
        pltpu.async_copy(ones_v, acc_sh.at[dst_all.at[j]], sem, add=True)

        @pl.when(j >= _DEG_WIN)
        def _():
            pltpu.make_async_copy(ones_v, acc_sh.at[dst_all.at[0]], sem).wait()

    @pl.loop(0, _DEG_WIN)
    def _(j):
        pltpu.make_async_copy(ones_v, acc_sh.at[dst_all.at[0]], sem).wait()

    plsc.subcore_barrier()
    pltpu.sync_copy(acc_sh.at[pl.ds(s * _ROWS_W, _ROWS_W)],
                    out_hbm.at[pl.ds(c * NPAD + s * _ROWS_W, _ROWS_W)])


_SP_NCH = 320
_SP_NBLK = 10
_SP_K = 4


@functools.partial(
    pl.kernel,
    out_type=[jax.ShapeDtypeStruct((NPAD, H), _F32),
              jax.ShapeDtypeStruct((NPAD, H), _F32)],
    mesh=_sc_mesh(),
    scratch_types=[
        pltpu.VMEM((80, _CH), jnp.int32),
        pltpu.VMEM((16, _CH), jnp.int32),
        pltpu.VMEM((8 * 32, H), _F32),
        pltpu.VMEM_SHARED((NPAD, H), _F32),
        pltpu.SemaphoreType.DMA,
        pltpu.SemaphoreType.DMA,
        pltpu.SemaphoreType.DMA,
    ],
)
def _sc_spmm(hs0_hbm, hs1_hbm, src_hbm, dst_hbm, zeros_hbm,
             out0_hbm, out1_hbm, dst_all, ring, rows, acc_sh,
             sem, sems, semi):
    c = lax.axis_index("c")
    s = lax.axis_index("s")
    pltpu.sync_copy(dst_hbm.at[s], dst_all)
    pltpu.sync_copy(zeros_hbm.at[pl.ds(s * _ROWS_W, _ROWS_W)],
                    acc_sh.at[pl.ds(s * _ROWS_W, _ROWS_W)])

    def _gather(j):
        b = j // 32
        rr = lax.rem(b, 2) * 8 + lax.rem(j, 32) // 4
        q = lax.rem(j, 4) * 32
        dstslc = rows.at[pl.ds(lax.rem(j, 8) * 32, 32)]

        @pl.when(c == 0)
        def _():
            pltpu.async_copy(hs0_hbm.at[ring.at[rr, pl.ds(q, 32)]],
                             dstslc, sem)

        @pl.when(c == 1)
        def _():
            pltpu.async_copy(hs1_hbm.at[ring.at[rr, pl.ds(q, 32)]],
                             dstslc, sem)

    def _gwait():
        pltpu.make_async_copy(hs0_hbm.at[ring.at[0, pl.ds(0, 32)]],
                              rows.at[pl.ds(0, 32)], sem).wait()

    def _swait():
        pltpu.make_async_copy(rows.at[pl.ds(0, 32)],
                              acc_sh.at[dst_all.at[0, pl.ds(0, 32)]],
                              sems).wait()

    pltpu.sync_copy(src_hbm.at[s].at[0], ring.at[pl.ds(0, 8)])
    pltpu.async_copy(src_hbm.at[s].at[1], ring.at[pl.ds(8, 8)], semi)
    plsc.subcore_barrier()
    for j in range(_SP_K):
        _gather(j)

    @pl.loop(0, _SP_NCH)
    def _(j):
        _gwait()
        pltpu.async_copy(rows.at[pl.ds(lax.rem(j, 8) * 32, 32)],
                         acc_sh.at[dst_all.at[j // 4,
                                              pl.ds(lax.rem(j, 4) * 32, 32)]],
                         sems, add=True)

        @pl.when(j >= _SP_K)
        def _():
            _swait()

        jn = j + _SP_K

        @pl.when(jn < _SP_NCH)
        def _():
            bn = jn // 32

            @pl.when(lax.rem(jn, 32) == 0)
            def _():
                pltpu.make_async_copy(src_hbm.at[s].at[0],
                                      ring.at[pl.ds(0, 8)], semi).wait()

                @pl.when(bn + 1 < _SP_NBLK)
                def _():
                    pltpu.async_copy(
                        src_hbm.at[s].at[bn + 1],
                        ring.at[pl.ds(lax.rem(bn + 1, 2) * 8, 8)], semi)

            _gather(jn)

    @pl.loop(0, _SP_K)
    def _(j):
        _swait()

    plsc.subcore_barrier()

    @pl.when(c == 0)
    def _():
        pltpu.sync_copy(acc_sh.at[pl.ds(s * _ROWS_W, _ROWS_W)],
                        out0_hbm.at[pl.ds(s * _ROWS_W, _ROWS_W)])

    @pl.when(c == 1)
    def _():
        pltpu.sync_copy(acc_sh.at[pl.ds(s * _ROWS_W, _ROWS_W)],
                        out1_hbm.at[pl.ds(s * _ROWS_W, _ROWS_W)])


def _dot(a, b, dims):
    return lax.dot_general(a, b, (dims, ((), ())),
                           precision=lax.Precision.HIGHEST,
                           preferred_element_type=_F32)


def _half_spec():
    return pl.BlockSpec((BN, H), lambda i: (i, 0))


def _tc_prep_body(deg0_ref, deg1_ref, x_ref, d_ref, xs0_ref, xs1_ref):
    deg = deg0_ref[0, :, 0:1] + deg1_ref[0, :, 0:1] + 1.0
    d = lax.rsqrt(deg)
    d_ref[...] = d
    xs = x_ref[...] * d
    xs0_ref[...] = xs[:, :H]
    xs1_ref[...] = xs[:, H:]


def _tc_prep(deg2, x):
    return pl.pallas_call(
        _tc_prep_body,
        grid=(NB,),
        in_specs=[
            pl.BlockSpec((1, BN, 16), lambda i: (0, i, 0)),
            pl.BlockSpec((1, BN, 16), lambda i: (1, i, 0)),
            pl.BlockSpec((BN, D), lambda i: (i, 0)),
        ],
        out_specs=[pl.BlockSpec((BN, 1), lambda i: (i, 0)),
                   _half_spec(), _half_spec()],
        out_shape=[jax.ShapeDtypeStruct((N, 1), _F32),
                   jax.ShapeDtypeStruct((NPAD, H), _F32),
                   jax.ShapeDtypeStruct((NPAD, H), _F32)],
    )(deg2, deg2, x)


def _tc_layer_body(t0_ref, t1_ref, h0_ref, h1_ref, d_ref, w_ref,
                   ys0_ref, ys1_ref):
    d = d_ref[...]
    u = jnp.concatenate(
        [t0_ref[...] + h0_ref[...], t1_ref[...] + h1_ref[...]], axis=1) * d
    y = jnp.maximum(_dot(u, w_ref[...], ((1,), (0,))), 0.0) * d
    ys0_ref[...] = y[:, :H]
    ys1_ref[...] = y[:, H:]


def _tc_layer(t0, t1, h0, h1, d, W):
    return pl.pallas_call(
        _tc_layer_body,
        grid=(NB,),
        in_specs=[_half_spec(), _half_spec(), _half_spec(), _half_spec(),
                  pl.BlockSpec((BN, 1), lambda i: (i, 0)),
                  pl.BlockSpec((D, D), lambda i: (0, 0))],
        out_specs=[_half_spec(), _half_spec()],
        out_shape=[jax.ShapeDtypeStruct((NPAD, H), _F32),
                   jax.ShapeDtypeStruct((NPAD, H), _F32)],
    )(t0, t1, h0, h1, d, W)


def _tc_final_body(t0_ref, t1_ref, h0_ref, h1_ref, d_ref, w_ref, z_ref):
    u = jnp.concatenate(
        [t0_ref[...] + h0_ref[...], t1_ref[...] + h1_ref[...]],
        axis=1) * d_ref[...]
    z_ref[...] = _dot(u, w_ref[...], ((1,), (0,)))


def _tc_final(t0, t1, h0, h1, d, W):
    return pl.pallas_call(
        _tc_final_body,
        grid=(NB,),
        in_specs=[_half_spec(), _half_spec(), _half_spec(), _half_spec(),
                  pl.BlockSpec((BN, 1), lambda i: (i, 0)),
                  pl.BlockSpec((D, D), lambda i: (0, 0))],
        out_specs=pl.BlockSpec((BN, D), lambda i: (i, 0)),
        out_shape=jax.ShapeDtypeStruct((N, D), _F32),
    )(t0, t1, h0, h1, d, W)


def _tc_cos_body(z_ref, o_ref):
    z = z_ref[...]
    zn = z * lax.rsqrt(jnp.sum(z * z, axis=1, keepdims=True))
    g = _dot(zn, zn, ((1,), (1,)))
    o_ref[...] = (jnp.sum(g) * (1.0 / (512.0 * 512.0))).reshape(1, 1)


def _tc_cos(z512):
    return pl.pallas_call(
        _tc_cos_body,
        out_shape=jax.ShapeDtypeStruct((1, 1), _F32),
    )(z512)


def kernel(x, edge_index, W1, W2, W3):
    npad_e = EPAD - E
    src = jnp.concatenate([edge_index[0], jnp.zeros((npad_e,), jnp.int32)])
    dst = jnp.concatenate([edge_index[1],
                           jnp.full((npad_e,), N, jnp.int32)])
    srcR = src.reshape(NS, _SP_NBLK, 8, _CH)
    dstR = dst.reshape(NS, 80, _CH)
    dst_degR = dst.reshape(NC * NS, _DEG_NCH, _CH)
    zeros16 = jnp.zeros((NPAD, 16), _F32)
    zerosH = jnp.zeros((NPAD, H), _F32)

    deg2 = _sc_degree(dst_degR, zeros16).reshape(NC, NPAD, 16)
    d, xs0, xs1 = _tc_prep(deg2, x)

    t0, t1 = _sc_spmm(xs0, xs1, srcR, dstR, zerosH)
    h0, h1 = _tc_layer(t0, t1, xs0, xs1, d, W1)

    t0, t1 = _sc_spmm(h0, h1, srcR, dstR, zerosH)
    g0, g1 = _tc_layer(t0, t1, h0, h1, d, W2)

    t0, t1 = _sc_spmm(g0, g1, srcR, dstR, zerosH)
    z = _tc_final(t0, t1, g0, g1, d, W3)

    corr = _tc_cos(z[:512])
    return z, corr[0, 0]
